# baseline (speedup 1.0000x reference)
import functools

import jax
import jax.numpy as jnp
from jax import lax
from jax.experimental import pallas as pl
from jax.experimental.pallas import tpu as pltpu
from jax.experimental.pallas import tpu_sc as plsc

USER_DIM = 32
MOVIE_DIM = 32
CAT_DIM = USER_DIM + MOVIE_DIM
OUT_W = 128
WAVE = 128
LANES = 16
DEPAD_ROWS = 8192


def _depad_body(in_ref, out_ref):
    x = in_ref[...]
    rows, d = x.shape
    pack = 128 // d
    for a in range(pack):
        out_ref[:, a * d:(a + 1) * d] = lax.slice(x, (a, 0), (rows, d),
                                                  (pack, 1))


def _depad(table):
    n, d = table.shape
    rows = min(DEPAD_ROWS, n)
    pack = 128 // d
    return pl.pallas_call(
        _depad_body,
        grid=(n // rows,),
        in_specs=[pl.BlockSpec((rows, d), lambda i: (i, 0))],
        out_specs=pl.BlockSpec((rows // pack, 128), lambda i: (i, 0)),
        out_shape=jax.ShapeDtypeStruct((n // pack, 128), jnp.float32),
        compiler_params=pltpu.CompilerParams(
            dimension_semantics=("arbitrary",)),
    )(table)


def _gather_body(b_per_w, nc, x1_hbm, x2_hbm, ue_hbm, me_hbm, out_hbm,
                 idx1_v, idx2_v, sub1_v, sub2_v,
                 grp_u, grp_m, buf, sem_u, sem_m, sem_w):
    wid = lax.axis_index("s") * nc + lax.axis_index("c")
    base = wid * b_per_w
    pltpu.sync_copy(x1_hbm.at[pl.ds(base, b_per_w)], idx1_v)
    pltpu.sync_copy(x2_hbm.at[pl.ds(base, b_per_w)], idx2_v)

    def split(k, carry):
        s = k * LANES
        i1 = idx1_v[pl.ds(s, LANES)]
        i2 = idx2_v[pl.ds(s, LANES)]
        sub1_v[pl.ds(s, LANES)] = lax.bitwise_and(i1, 3)
        sub2_v[pl.ds(s, LANES)] = lax.bitwise_and(i2, 3)
        idx1_v[pl.ds(s, LANES)] = lax.shift_right_logical(i1, 2)
        idx2_v[pl.ds(s, LANES)] = lax.shift_right_logical(i2, 2)
        return carry

    lax.fori_loop(0, b_per_w // LANES, split, 0)

    iota = lax.iota(jnp.int32, LANES)

    def issue(w, slot):
        jbase = w * WAVE
        pltpu.async_copy(ue_hbm.at[idx1_v.at[pl.ds(jbase, WAVE)]],
                         grp_u.at[slot], sem_u.at[slot])
        pltpu.async_copy(me_hbm.at[idx2_v.at[pl.ds(jbase, WAVE)]],
                         grp_m.at[slot], sem_m.at[slot])

    def extract(grp, sub_v, jbase, slot, col0):
        def one(jl, carry):
            jv = jnp.full((LANES,), jbase + jl, jnp.int32)
            subj = plsc.load_gather(sub_v, [jv])
            col = subj * USER_DIM + iota
            jlv = jnp.full((LANES,), jl, jnp.int32)
            lo = plsc.load_gather(grp, [jlv, col])
            hi = plsc.load_gather(grp, [jlv, col + LANES])
            buf[slot, jl, pl.ds(col0, LANES)] = lo
            buf[slot, jl, pl.ds(col0 + LANES, LANES)] = hi
            return carry

        lax.fori_loop(0, WAVE, one, 0)

    def process(w, slot):
        @pl.when(w >= 2)
        def _():
            pltpu.make_async_copy(out_hbm.at[pl.ds(0, WAVE)],
                                  buf.at[slot], sem_w.at[slot]).wait()

        jbase = w * WAVE
        pltpu.make_async_copy(ue_hbm.at[pl.ds(0, WAVE)], grp_u.at[slot],
                              sem_u.at[slot]).wait()
        extract(grp_u.at[slot], sub1_v, jbase, slot, 0)
        pltpu.make_async_copy(me_hbm.at[pl.ds(0, WAVE)], grp_m.at[slot],
                              sem_m.at[slot]).wait()
        extract(grp_m.at[slot], sub2_v, jbase, slot, USER_DIM)
        pltpu.async_copy(buf.at[slot],
                         out_hbm.at[pl.ds(base + jbase, WAVE)],
                         sem_w.at[slot])

    n_waves = b_per_w // WAVE
    issue(0, 0)

    def wave(w, carry):
        slot = lax.rem(w, 2)
        nslot = lax.rem(w + 1, 2)
        issue(w + 1, nslot)
        process(w, slot)
        return carry

    lax.fori_loop(0, n_waves - 1, wave, 0)
    process(n_waves - 1, lax.rem(n_waves - 1, 2))

    pltpu.make_async_copy(out_hbm.at[pl.ds(0, WAVE)], buf.at[0],
                          sem_w.at[0]).wait()
    pltpu.make_async_copy(out_hbm.at[pl.ds(0, WAVE)], buf.at[1],
                          sem_w.at[1]).wait()


def _mlp_body(x_ref, w1_ref, b1_ref, w2_ref, b2_ref, o_ref):
    x = x_ref[...][:, :CAT_DIM]
    h = jnp.dot(x, w1_ref[...],
                preferred_element_type=jnp.float32) + b1_ref[...]
    h = jnp.maximum(h, 0.0)
    o = jnp.dot(h, w2_ref[...],
                preferred_element_type=jnp.float32) + b2_ref[...]
    o_ref[...] = jax.nn.sigmoid(o)


def kernel(x1, x2, user_embed, movie_embed, fc1_w, fc1_b, fc2_w, fc2_b):
    B = x1.shape[0]
    info = plsc.get_sparse_core_info()
    nc, ns = info.num_cores, info.num_subcores
    nw = nc * ns
    b_per_w = B // nw

    x1i = x1.astype(jnp.int32)
    x2i = x2.astype(jnp.int32)
    nu, nm = user_embed.shape[0], movie_embed.shape[0]
    ue2 = user_embed.reshape(nu // 4, 128)
    me2 = movie_embed.reshape(nm // 4, 128)

    gather = pl.kernel(
        functools.partial(_gather_body, b_per_w, nc),
        out_type=jax.ShapeDtypeStruct((B, OUT_W), jnp.float32),
        mesh=plsc.VectorSubcoreMesh(core_axis_name="c", subcore_axis_name="s"),
        scratch_types=[
            pltpu.VMEM((b_per_w,), jnp.int32),
            pltpu.VMEM((b_per_w,), jnp.int32),
            pltpu.VMEM((b_per_w,), jnp.int32),
            pltpu.VMEM((b_per_w,), jnp.int32),
            pltpu.VMEM((2, WAVE, 128), jnp.float32),
            pltpu.VMEM((2, WAVE, 128), jnp.float32),
            pltpu.VMEM((2, WAVE, OUT_W), jnp.float32),
            pltpu.SemaphoreType.DMA((2,)),
            pltpu.SemaphoreType.DMA((2,)),
            pltpu.SemaphoreType.DMA((2,)),
        ],
        compiler_params=pltpu.CompilerParams(needs_layout_passes=False),
    )
    x = gather(x1i, x2i, ue2, me2)

    hidden = fc1_w.shape[0]
    hp = 128
    w1t = jnp.zeros((CAT_DIM, hp), jnp.float32).at[:, :hidden].set(fc1_w.T)
    b1 = jnp.zeros((1, hp), jnp.float32).at[:, :hidden].set(fc1_b[None, :])
    w2t = jnp.zeros((hp, 1), jnp.float32).at[:hidden, :].set(fc2_w.T)
    b2 = fc2_b.reshape(1, 1)

    blk = 2048
    grid = (B // blk,)
    out = pl.pallas_call(
        _mlp_body,
        grid=grid,
        in_specs=[
            pl.BlockSpec((blk, OUT_W), lambda i: (i, 0)),
            pl.BlockSpec((CAT_DIM, hp), lambda i: (0, 0)),
            pl.BlockSpec((1, hp), lambda i: (0, 0)),
            pl.BlockSpec((hp, 1), lambda i: (0, 0)),
            pl.BlockSpec((1, 1), lambda i: (0, 0)),
        ],
        out_specs=pl.BlockSpec((blk, 1), lambda i: (i, 0)),
        out_shape=jax.ShapeDtypeStruct((B, 1), jnp.float32),
        compiler_params=pltpu.CompilerParams(
            dimension_semantics=("arbitrary",)),
    )(x, w1t, b1, w2t, b2)
    return out


# speedup vs baseline: 2.3609x; 2.3609x over previous
import functools

import jax
import jax.numpy as jnp
from jax import lax
from jax.experimental import pallas as pl
from jax.experimental.pallas import tpu as pltpu
from jax.experimental.pallas import tpu_sc as plsc

USER_DIM = 32
MOVIE_DIM = 32
CAT_DIM = USER_DIM + MOVIE_DIM
OUT_W = 128
WAVE = 32
LANES = 16


def _gather_body(b_per_w, nc, x1_hbm, x2_hbm, ue_hbm, me_hbm, out_hbm,
                 idx1_v, idx2_v, sub1_v, sub2_v,
                 grp_u, grp_m, buf, sem_u, sem_m, sem_w):
    wid = lax.axis_index("s") * nc + lax.axis_index("c")
    base = wid * b_per_w
    pltpu.sync_copy(x1_hbm.at[pl.ds(base, b_per_w)], idx1_v)
    pltpu.sync_copy(x2_hbm.at[pl.ds(base, b_per_w)], idx2_v)

    def split(k, carry):
        s = k * LANES
        i1 = idx1_v[pl.ds(s, LANES)]
        i2 = idx2_v[pl.ds(s, LANES)]
        sub1_v[pl.ds(s, LANES)] = lax.bitwise_and(i1, 3)
        sub2_v[pl.ds(s, LANES)] = lax.bitwise_and(i2, 3)
        idx1_v[pl.ds(s, LANES)] = lax.shift_right_logical(i1, 2)
        idx2_v[pl.ds(s, LANES)] = lax.shift_right_logical(i2, 2)
        return carry

    lax.fori_loop(0, b_per_w // LANES, split, 0)

    iota = lax.iota(jnp.int32, LANES)

    def issue(w, slot):
        jbase = w * WAVE
        for half in range(WAVE // LANES):
            hb = jbase + half * LANES
            i1 = idx1_v[pl.ds(hb, LANES)]
            for t in range(LANES):
                pltpu.async_copy(ue_hbm.at[i1[t]],
                                 grp_u.at[slot, half * LANES + t],
                                 sem_u.at[slot])
        for half in range(WAVE // LANES):
            hb = jbase + half * LANES
            i2 = idx2_v[pl.ds(hb, LANES)]
            for t in range(LANES):
                pltpu.async_copy(me_hbm.at[i2[t]],
                                 grp_m.at[slot, half * LANES + t],
                                 sem_m.at[slot])

    def extract(grp, sub_v, jbase, slot, col0):
        def one(jl, carry):
            jv = jnp.full((LANES,), jbase + jl, jnp.int32)
            subj = plsc.load_gather(sub_v, [jv])
            jlv = jnp.full((LANES,), jl, jnp.int32)
            lo = plsc.load_gather(grp, [jlv, subj, iota])
            hi = plsc.load_gather(grp, [jlv, subj, iota + LANES])
            buf[slot, jl, pl.ds(col0, LANES)] = lo
            buf[slot, jl, pl.ds(col0 + LANES, LANES)] = hi
            return carry

        lax.fori_loop(0, WAVE, one, 0)

    def process(w, slot):
        @pl.when(w >= 2)
        def _():
            pltpu.make_async_copy(out_hbm.at[pl.ds(0, WAVE)],
                                  buf.at[slot], sem_w.at[slot]).wait()

        jbase = w * WAVE
        pltpu.make_async_copy(ue_hbm.at[pl.ds(0, WAVE)], grp_u.at[slot],
                              sem_u.at[slot]).wait()
        extract(grp_u.at[slot], sub1_v, jbase, slot, 0)
        pltpu.make_async_copy(ue_hbm.at[pl.ds(0, WAVE)], grp_m.at[slot],
                              sem_m.at[slot]).wait()
        extract(grp_m.at[slot], sub2_v, jbase, slot, USER_DIM)
        pltpu.async_copy(buf.at[slot],
                         out_hbm.at[pl.ds(base + jbase, WAVE)],
                         sem_w.at[slot])

    n_waves = b_per_w // WAVE
    issue(0, 0)

    def wave(w, carry):
        slot = lax.rem(w, 2)
        nslot = lax.rem(w + 1, 2)
        issue(w + 1, nslot)
        process(w, slot)
        return carry

    lax.fori_loop(0, n_waves - 1, wave, 0)
    process(n_waves - 1, lax.rem(n_waves - 1, 2))

    pltpu.make_async_copy(out_hbm.at[pl.ds(0, WAVE)], buf.at[0],
                          sem_w.at[0]).wait()
    pltpu.make_async_copy(out_hbm.at[pl.ds(0, WAVE)], buf.at[1],
                          sem_w.at[1]).wait()


def _mlp_body(x_ref, w1_ref, b1_ref, w2_ref, b2_ref, o_ref):
    x = x_ref[...][:, :CAT_DIM]
    h = jnp.dot(x, w1_ref[...],
                preferred_element_type=jnp.float32) + b1_ref[...]
    h = jnp.maximum(h, 0.0)
    o = jnp.dot(h, w2_ref[...],
                preferred_element_type=jnp.float32) + b2_ref[...]
    o_ref[...] = jax.nn.sigmoid(o)


def kernel(x1, x2, user_embed, movie_embed, fc1_w, fc1_b, fc2_w, fc2_b):
    B = x1.shape[0]
    info = plsc.get_sparse_core_info()
    nc, ns = info.num_cores, info.num_subcores
    nw = nc * ns
    b_per_w = B // nw

    x1i = x1.astype(jnp.int32)
    x2i = x2.astype(jnp.int32)
    nu, nm = user_embed.shape[0], movie_embed.shape[0]
    ue3 = user_embed.reshape(nu // 4, 4, USER_DIM)
    me3 = movie_embed.reshape(nm // 4, 4, MOVIE_DIM)

    gather = pl.kernel(
        functools.partial(_gather_body, b_per_w, nc),
        out_type=jax.ShapeDtypeStruct((B, OUT_W), jnp.float32),
        mesh=plsc.VectorSubcoreMesh(core_axis_name="c", subcore_axis_name="s"),
        scratch_types=[
            pltpu.VMEM((b_per_w,), jnp.int32),
            pltpu.VMEM((b_per_w,), jnp.int32),
            pltpu.VMEM((b_per_w,), jnp.int32),
            pltpu.VMEM((b_per_w,), jnp.int32),
            pltpu.VMEM((2, WAVE, 4, USER_DIM), jnp.float32),
            pltpu.VMEM((2, WAVE, 4, MOVIE_DIM), jnp.float32),
            pltpu.VMEM((2, WAVE, OUT_W), jnp.float32),
            pltpu.SemaphoreType.DMA((2,)),
            pltpu.SemaphoreType.DMA((2,)),
            pltpu.SemaphoreType.DMA((2,)),
        ],
        compiler_params=pltpu.CompilerParams(needs_layout_passes=False),
    )
    x = gather(x1i, x2i, ue3, me3)

    hidden = fc1_w.shape[0]
    hp = 128
    w1t = jnp.zeros((CAT_DIM, hp), jnp.float32).at[:, :hidden].set(fc1_w.T)
    b1 = jnp.zeros((1, hp), jnp.float32).at[:, :hidden].set(fc1_b[None, :])
    w2t = jnp.zeros((hp, 1), jnp.float32).at[:hidden, :].set(fc2_w.T)
    b2 = fc2_b.reshape(1, 1)

    blk = 2048
    grid = (B // blk,)
    out = pl.pallas_call(
        _mlp_body,
        grid=grid,
        in_specs=[
            pl.BlockSpec((blk, OUT_W), lambda i: (i, 0)),
            pl.BlockSpec((CAT_DIM, hp), lambda i: (0, 0)),
            pl.BlockSpec((1, hp), lambda i: (0, 0)),
            pl.BlockSpec((hp, 1), lambda i: (0, 0)),
            pl.BlockSpec((1, 1), lambda i: (0, 0)),
        ],
        out_specs=pl.BlockSpec((blk, 1), lambda i: (i, 0)),
        out_shape=jax.ShapeDtypeStruct((B, 1), jnp.float32),
        compiler_params=pltpu.CompilerParams(
            dimension_semantics=("arbitrary",)),
    )(x, w1t, b1, w2t, b2)
    return out


# speedup vs baseline: 2.3634x; 1.0011x over previous
import functools

import jax
import jax.numpy as jnp
from jax import lax
from jax.experimental import pallas as pl
from jax.experimental.pallas import tpu as pltpu
from jax.experimental.pallas import tpu_sc as plsc

USER_DIM = 32
MOVIE_DIM = 32
CAT_DIM = USER_DIM + MOVIE_DIM
OUT_W = 128
WAVE = 32
NSLOT = 3
LANES = 16


def _gather_body(b_per_w, nc, x1_hbm, x2_hbm, ue_hbm, me_hbm, out_hbm,
                 idx1_v, idx2_v, sub1_v, sub2_v,
                 grp_u, grp_m, buf, sem_u, sem_m, sem_w):
    wid = lax.axis_index("s") * nc + lax.axis_index("c")
    base = wid * b_per_w
    pltpu.sync_copy(x1_hbm.at[pl.ds(base, b_per_w)], idx1_v)
    pltpu.sync_copy(x2_hbm.at[pl.ds(base, b_per_w)], idx2_v)

    def split(k, carry):
        s = k * LANES
        i1 = idx1_v[pl.ds(s, LANES)]
        i2 = idx2_v[pl.ds(s, LANES)]
        sub1_v[pl.ds(s, LANES)] = lax.bitwise_and(i1, 3)
        sub2_v[pl.ds(s, LANES)] = lax.bitwise_and(i2, 3)
        idx1_v[pl.ds(s, LANES)] = lax.shift_right_logical(i1, 2)
        idx2_v[pl.ds(s, LANES)] = lax.shift_right_logical(i2, 2)
        return carry

    lax.fori_loop(0, b_per_w // LANES, split, 0)

    iota = lax.iota(jnp.int32, LANES)

    def issue(w, slot):
        jbase = w * WAVE
        for half in range(WAVE // LANES):
            hb = jbase + half * LANES
            i1 = idx1_v[pl.ds(hb, LANES)]
            for t in range(LANES):
                pltpu.async_copy(ue_hbm.at[i1[t]],
                                 grp_u.at[slot, half * LANES + t],
                                 sem_u.at[slot])
        for half in range(WAVE // LANES):
            hb = jbase + half * LANES
            i2 = idx2_v[pl.ds(hb, LANES)]
            for t in range(LANES):
                pltpu.async_copy(me_hbm.at[i2[t]],
                                 grp_m.at[slot, half * LANES + t],
                                 sem_m.at[slot])

    def extract(grp, sub_v, jbase, slot, col0):
        def one(jl, carry):
            jv = jnp.full((LANES,), jbase + jl, jnp.int32)
            subj = plsc.load_gather(sub_v, [jv])
            jlv = jnp.full((LANES,), jl, jnp.int32)
            lo = plsc.load_gather(grp, [jlv, subj, iota])
            hi = plsc.load_gather(grp, [jlv, subj, iota + LANES])
            buf[slot, jl, pl.ds(col0, LANES)] = lo
            buf[slot, jl, pl.ds(col0 + LANES, LANES)] = hi
            return carry

        lax.fori_loop(0, WAVE, one, 0)

    def process(w, slot):
        @pl.when(w >= NSLOT)
        def _():
            pltpu.make_async_copy(out_hbm.at[pl.ds(0, WAVE)],
                                  buf.at[slot], sem_w.at[slot]).wait()

        jbase = w * WAVE
        pltpu.make_async_copy(ue_hbm.at[pl.ds(0, WAVE)], grp_u.at[slot],
                              sem_u.at[slot]).wait()
        extract(grp_u.at[slot], sub1_v, jbase, slot, 0)
        pltpu.make_async_copy(ue_hbm.at[pl.ds(0, WAVE)], grp_m.at[slot],
                              sem_m.at[slot]).wait()
        extract(grp_m.at[slot], sub2_v, jbase, slot, USER_DIM)
        pltpu.async_copy(buf.at[slot],
                         out_hbm.at[pl.ds(base + jbase, WAVE)],
                         sem_w.at[slot])

    n_waves = b_per_w // WAVE
    for p in range(NSLOT - 1):
        issue(p, p)

    def wave(w, carry):
        slot = lax.rem(w, NSLOT)
        nslot = lax.rem(w + NSLOT - 1, NSLOT)
        issue(w + NSLOT - 1, nslot)
        process(w, slot)
        return carry

    lax.fori_loop(0, n_waves - (NSLOT - 1), wave, 0)
    for w in range(n_waves - (NSLOT - 1), n_waves):
        process(w, w % NSLOT)

    for s in range(NSLOT):
        pltpu.make_async_copy(out_hbm.at[pl.ds(0, WAVE)], buf.at[s],
                              sem_w.at[s]).wait()


def _mlp_body(x_ref, w1_ref, b1_ref, w2_ref, b2_ref, o_ref):
    x = x_ref[...][:, :CAT_DIM]
    h = jnp.dot(x, w1_ref[...],
                preferred_element_type=jnp.float32) + b1_ref[...]
    h = jnp.maximum(h, 0.0)
    o = jnp.dot(h, w2_ref[...],
                preferred_element_type=jnp.float32) + b2_ref[...]
    o_ref[...] = jax.nn.sigmoid(o)


def kernel(x1, x2, user_embed, movie_embed, fc1_w, fc1_b, fc2_w, fc2_b):
    B = x1.shape[0]
    info = plsc.get_sparse_core_info()
    nc, ns = info.num_cores, info.num_subcores
    nw = nc * ns
    b_per_w = B // nw

    x1i = x1.astype(jnp.int32)
    x2i = x2.astype(jnp.int32)
    nu, nm = user_embed.shape[0], movie_embed.shape[0]
    ue3 = user_embed.reshape(nu // 4, 4, USER_DIM)
    me3 = movie_embed.reshape(nm // 4, 4, MOVIE_DIM)

    gather = pl.kernel(
        functools.partial(_gather_body, b_per_w, nc),
        out_type=jax.ShapeDtypeStruct((B, OUT_W), jnp.float32),
        mesh=plsc.VectorSubcoreMesh(core_axis_name="c", subcore_axis_name="s"),
        scratch_types=[
            pltpu.VMEM((b_per_w,), jnp.int32),
            pltpu.VMEM((b_per_w,), jnp.int32),
            pltpu.VMEM((b_per_w,), jnp.int32),
            pltpu.VMEM((b_per_w,), jnp.int32),
            pltpu.VMEM((NSLOT, WAVE, 4, USER_DIM), jnp.float32),
            pltpu.VMEM((NSLOT, WAVE, 4, MOVIE_DIM), jnp.float32),
            pltpu.VMEM((NSLOT, WAVE, OUT_W), jnp.float32),
            pltpu.SemaphoreType.DMA((NSLOT,)),
            pltpu.SemaphoreType.DMA((NSLOT,)),
            pltpu.SemaphoreType.DMA((NSLOT,)),
        ],
        compiler_params=pltpu.CompilerParams(needs_layout_passes=False),
    )
    x = gather(x1i, x2i, ue3, me3)

    hidden = fc1_w.shape[0]
    hp = 128
    w1t = jnp.zeros((CAT_DIM, hp), jnp.float32).at[:, :hidden].set(fc1_w.T)
    b1 = jnp.zeros((1, hp), jnp.float32).at[:, :hidden].set(fc1_b[None, :])
    w2t = jnp.zeros((hp, 1), jnp.float32).at[:hidden, :].set(fc2_w.T)
    b2 = fc2_b.reshape(1, 1)

    blk = 2048
    grid = (B // blk,)
    out = pl.pallas_call(
        _mlp_body,
        grid=grid,
        in_specs=[
            pl.BlockSpec((blk, OUT_W), lambda i: (i, 0)),
            pl.BlockSpec((CAT_DIM, hp), lambda i: (0, 0)),
            pl.BlockSpec((1, hp), lambda i: (0, 0)),
            pl.BlockSpec((hp, 1), lambda i: (0, 0)),
            pl.BlockSpec((1, 1), lambda i: (0, 0)),
        ],
        out_specs=pl.BlockSpec((blk, 1), lambda i: (i, 0)),
        out_shape=jax.ShapeDtypeStruct((B, 1), jnp.float32),
        compiler_params=pltpu.CompilerParams(
            dimension_semantics=("arbitrary",)),
    )(x, w1t, b1, w2t, b2)
    return out
